# baseline (speedup 1.0000x reference)
import dataclasses

import jax
import jax.numpy as jnp
from jax import lax
from jax.experimental import pallas as pl
from jax.experimental.pallas import tpu as pltpu
from jax.experimental.pallas import tpu_sc as plsc

N = 10000
D = 128
E = 320000
NC = 2
NS = 16
NW = NC * NS
CH = 125
NCHUNK = E // (NW * CH)
EROWS = E // CH
NP = 10240
RPT = NP // NS

_MESH = plsc.VectorSubcoreMesh(core_axis_name="c", subcore_axis_name="s")
EPT = E // NW

_CP = pltpu.CompilerParams()
if "needs_layout_passes" in pltpu.CompilerParams.__dataclass_fields__:
    _CP = dataclasses.replace(_CP, needs_layout_passes=False)


def _hist_body(dst_hbm, out_hbm, idx_v, hist):
    c = lax.axis_index("c")
    s = lax.axis_index("s")
    wid = c * NS + s

    @pl.loop(0, NP // 16)
    def _(k):
        hist[0, pl.ds(k * 16, 16)] = jnp.zeros((16,), jnp.float32)

    pltpu.sync_copy(dst_hbm.at[pl.ds(wid * EPT, EPT)], idx_v)
    zeros_i = jnp.zeros((16,), jnp.int32)
    ones_f = jnp.ones((16,), jnp.float32)

    @pl.loop(0, EPT // 16)
    def _(k):
        iv = idx_v[pl.ds(k * 16, 16)]
        plsc.addupdate_scatter(hist, [zeros_i, iv], ones_f)

    pltpu.sync_copy(hist, out_hbm.at[wid])


_hist = pl.kernel(
    _hist_body,
    out_type=jax.ShapeDtypeStruct((NW, 1, NP), jnp.float32),
    mesh=_MESH,
    compiler_params=_CP,
    scratch_types=[
        pltpu.VMEM((EPT,), jnp.int32),
        pltpu.VMEM((1, NP), jnp.float32),
    ],
)


def _agg_body(hs_hbm, src_hbm, dst_hbm, zero_hbm, out_hbm,
              dst_v, s0, s1, r0, r1, acc, gsem0, gsem1, isem0, isem1):
    c = lax.axis_index("c")
    s = lax.axis_index("s")
    wid = c * NS + s
    pltpu.sync_copy(zero_hbm, acc.at[pl.ds(s * RPT, RPT)])
    pltpu.sync_copy(dst_hbm.at[pl.ds(wid * NCHUNK, NCHUNK)], dst_v)
    plsc.subcore_barrier()

    pltpu.sync_copy(src_hbm.at[wid, 0], s0)
    pltpu.sync_copy(src_hbm.at[wid, 1], s1)
    pltpu.async_copy(hs_hbm.at[s0.at[0]], r0, gsem0)
    pltpu.async_copy(hs_hbm.at[s1.at[0]], r1, gsem1)

    @pl.loop(0, NCHUNK - 2, step=2)
    def _(j):
        pltpu.make_async_copy(hs_hbm.at[s0.at[0]], r0, gsem0).wait()
        pltpu.async_copy(src_hbm.at[wid, j + 2], s0, isem0)
        pltpu.sync_copy(r0, acc.at[dst_v.at[j]], add=True)
        pltpu.make_async_copy(src_hbm.at[wid, j + 2], s0, isem0).wait()
        pltpu.async_copy(hs_hbm.at[s0.at[0]], r0, gsem0)
        pltpu.make_async_copy(hs_hbm.at[s1.at[0]], r1, gsem1).wait()
        pltpu.async_copy(src_hbm.at[wid, j + 3], s1, isem1)
        pltpu.sync_copy(r1, acc.at[dst_v.at[j + 1]], add=True)
        pltpu.make_async_copy(src_hbm.at[wid, j + 3], s1, isem1).wait()
        pltpu.async_copy(hs_hbm.at[s1.at[0]], r1, gsem1)

    pltpu.make_async_copy(hs_hbm.at[s0.at[0]], r0, gsem0).wait()
    pltpu.sync_copy(r0, acc.at[dst_v.at[NCHUNK - 2]], add=True)
    pltpu.make_async_copy(hs_hbm.at[s1.at[0]], r1, gsem1).wait()
    pltpu.sync_copy(r1, acc.at[dst_v.at[NCHUNK - 1]], add=True)

    plsc.subcore_barrier()
    pltpu.sync_copy(acc.at[pl.ds(s * RPT, RPT)],
                    out_hbm.at[c, pl.ds(s * RPT, RPT)])


_agg = pl.kernel(
    _agg_body,
    out_type=jax.ShapeDtypeStruct((NC, NP, D), jnp.float32),
    mesh=_MESH,
    scratch_types=[
        pltpu.VMEM((NCHUNK, CH), jnp.int32),
        pltpu.VMEM((1, CH), jnp.int32),
        pltpu.VMEM((1, CH), jnp.int32),
        pltpu.VMEM((CH, D), jnp.float32),
        pltpu.VMEM((CH, D), jnp.float32),
        pltpu.VMEM_SHARED((NP, D), jnp.float32),
        pltpu.SemaphoreType.DMA,
        pltpu.SemaphoreType.DMA,
        pltpu.SemaphoreType.DMA,
        pltpu.SemaphoreType.DMA,
    ],
)


BR = 512
GRID = NP // BR


def _dot(a, b):
    return jax.lax.dot(a, b, precision=jax.lax.Precision.HIGHEST,
                       preferred_element_type=jnp.float32)


def _dinv_col(degp):
    deg = jnp.sum(degp, axis=(0, 1))
    dinv = lax.rsqrt(deg + 1.0)[None, :]
    eye = (lax.broadcasted_iota(jnp.int32, (BR, BR), 0)
           == lax.broadcasted_iota(jnp.int32, (BR, BR), 1))
    return _dot(eye.astype(jnp.float32) * dinv, jnp.ones((BR, 1), jnp.float32))


def _mm_body(x_ref, w_ref, h_ref):
    h_ref[...] = _dot(x_ref[...], w_ref[...])


_mm = pl.pallas_call(
    _mm_body,
    grid=(GRID,),
    in_specs=[
        pl.BlockSpec((BR, D), lambda i: (i, 0)),
        pl.BlockSpec((D, D), lambda i: (0, 0)),
    ],
    out_specs=pl.BlockSpec((BR, D), lambda i: (i, 0)),
    out_shape=jax.ShapeDtypeStruct((NP, D), jnp.float32),
)


def _scale_body(h_ref, degp_ref, hs_ref, dinv_ref):
    dinv = _dinv_col(degp_ref[...])
    dinv_ref[...] = dinv
    hs_ref[...] = h_ref[...] * dinv


_scale = pl.pallas_call(
    _scale_body,
    grid=(GRID,),
    in_specs=[
        pl.BlockSpec((BR, D), lambda i: (i, 0)),
        pl.BlockSpec((NW, 1, BR), lambda i: (0, 0, i)),
    ],
    out_specs=[pl.BlockSpec((BR, D), lambda i: (i, 0)),
               pl.BlockSpec((BR, 1), lambda i: (i, 0))],
    out_shape=[jax.ShapeDtypeStruct((NP, D), jnp.float32),
               jax.ShapeDtypeStruct((NP, 1), jnp.float32)],
)


def _mid_body(accp_ref, hs_ref, dinv_ref, b_ref, w_ref, hs2_ref):
    dinv = dinv_ref[...]
    z = (accp_ref[0] + accp_ref[1] + hs_ref[...]) * dinv + b_ref[...]
    z = jnp.maximum(z, 0.0)
    hs2_ref[...] = _dot(z, w_ref[...]) * dinv


_mid = pl.pallas_call(
    _mid_body,
    grid=(GRID,),
    in_specs=[
        pl.BlockSpec((NC, BR, D), lambda i: (0, i, 0)),
        pl.BlockSpec((BR, D), lambda i: (i, 0)),
        pl.BlockSpec((BR, 1), lambda i: (i, 0)),
        pl.BlockSpec((1, D), lambda i: (0, 0)),
        pl.BlockSpec((D, D), lambda i: (0, 0)),
    ],
    out_specs=pl.BlockSpec((BR, D), lambda i: (i, 0)),
    out_shape=jax.ShapeDtypeStruct((NP, D), jnp.float32),
)


def _final_body(accp_ref, hs_ref, dinv_ref, b_ref, out_ref):
    out_ref[...] = ((accp_ref[0] + accp_ref[1] + hs_ref[...]) * dinv_ref[...]
                    + b_ref[...])


_final = pl.pallas_call(
    _final_body,
    grid=(GRID,),
    in_specs=[
        pl.BlockSpec((NC, BR, D), lambda i: (0, i, 0)),
        pl.BlockSpec((BR, D), lambda i: (i, 0)),
        pl.BlockSpec((BR, 1), lambda i: (i, 0)),
        pl.BlockSpec((1, D), lambda i: (0, 0)),
    ],
    out_specs=pl.BlockSpec((BR, D), lambda i: (i, 0)),
    out_shape=jax.ShapeDtypeStruct((N, D), jnp.float32),
)


def kernel(x, edge_index, W1, b1, W2, b2):
    xp = jnp.pad(x, ((0, NP - N), (0, 0)))
    src4 = edge_index[0].astype(jnp.int32).reshape(NW, NCHUNK, 1, CH)
    dst1 = edge_index[1].astype(jnp.int32)
    dst2 = dst1.reshape(EROWS, CH)
    zeroD = jnp.zeros((RPT, D), jnp.float32)
    b1r = b1.reshape(1, D)
    b2r = b2.reshape(1, D)

    h1r = _mm(xp, W1)
    degp = _hist(dst1)
    hs1, dinv = _scale(h1r, degp)
    acc1 = _agg(hs1, src4, dst2, zeroD)
    hs2 = _mid(acc1, hs1, dinv, b1r, W2)
    acc2 = _agg(hs2, src4, dst2, zeroD)
    return _final(acc2, hs2, dinv, b2r)


# speedup vs baseline: 1.0421x; 1.0421x over previous
import dataclasses

import jax
import jax.numpy as jnp
from jax import lax
from jax.experimental import pallas as pl
from jax.experimental.pallas import tpu as pltpu
from jax.experimental.pallas import tpu_sc as plsc

N = 10000
D = 128
E = 320000
NC = 2
NS = 16
NW = NC * NS
CH = 125
NCHUNK = E // (NW * CH)
EROWS = E // CH
NP = 10240
RPT = NP // NS

_MESH = plsc.VectorSubcoreMesh(core_axis_name="c", subcore_axis_name="s")
EPT = E // NW

_CP = pltpu.CompilerParams()
if "needs_layout_passes" in pltpu.CompilerParams.__dataclass_fields__:
    _CP = dataclasses.replace(_CP, needs_layout_passes=False)


def _hist_body(dst_hbm, out_hbm, idx_v, hist):
    c = lax.axis_index("c")
    s = lax.axis_index("s")
    wid = c * NS + s

    @pl.loop(0, NP // 64)
    def _(k):
        for u in range(4):
            hist[0, pl.ds(k * 64 + u * 16, 16)] = jnp.zeros((16,), jnp.float32)

    pltpu.sync_copy(dst_hbm.at[pl.ds(wid * EPT, EPT)], idx_v)
    zeros_i = jnp.zeros((16,), jnp.int32)
    ones_f = jnp.ones((16,), jnp.float32)

    @pl.loop(0, EPT // 80)
    def _(k):
        for u in range(5):
            iv = idx_v[pl.ds(k * 80 + u * 16, 16)]
            plsc.addupdate_scatter(hist, [zeros_i, iv], ones_f)

    pltpu.sync_copy(hist, out_hbm.at[wid])


_hist = pl.kernel(
    _hist_body,
    out_type=jax.ShapeDtypeStruct((NW, 1, NP), jnp.float32),
    mesh=_MESH,
    compiler_params=_CP,
    scratch_types=[
        pltpu.VMEM((EPT,), jnp.int32),
        pltpu.VMEM((1, NP), jnp.float32),
    ],
)


def _agg_body(hs_hbm, src_hbm, dst_hbm, zero_hbm, out_hbm,
              dst_v, s0, s1, r0, r1, acc, gsem0, gsem1, isem0, isem1):
    c = lax.axis_index("c")
    s = lax.axis_index("s")
    wid = c * NS + s
    pltpu.sync_copy(zero_hbm, acc.at[pl.ds(s * RPT, RPT)])
    pltpu.sync_copy(dst_hbm.at[pl.ds(wid * NCHUNK, NCHUNK)], dst_v)
    plsc.subcore_barrier()

    pltpu.sync_copy(src_hbm.at[wid, 0], s0)
    pltpu.sync_copy(src_hbm.at[wid, 1], s1)
    pltpu.async_copy(hs_hbm.at[s0.at[0]], r0, gsem0)
    pltpu.async_copy(hs_hbm.at[s1.at[0]], r1, gsem1)

    @pl.loop(0, NCHUNK - 2, step=2)
    def _(j):
        pltpu.make_async_copy(hs_hbm.at[s0.at[0]], r0, gsem0).wait()
        pltpu.async_copy(src_hbm.at[wid, j + 2], s0, isem0)
        pltpu.sync_copy(r0, acc.at[dst_v.at[j]], add=True)
        pltpu.make_async_copy(src_hbm.at[wid, j + 2], s0, isem0).wait()
        pltpu.async_copy(hs_hbm.at[s0.at[0]], r0, gsem0)
        pltpu.make_async_copy(hs_hbm.at[s1.at[0]], r1, gsem1).wait()
        pltpu.async_copy(src_hbm.at[wid, j + 3], s1, isem1)
        pltpu.sync_copy(r1, acc.at[dst_v.at[j + 1]], add=True)
        pltpu.make_async_copy(src_hbm.at[wid, j + 3], s1, isem1).wait()
        pltpu.async_copy(hs_hbm.at[s1.at[0]], r1, gsem1)

    pltpu.make_async_copy(hs_hbm.at[s0.at[0]], r0, gsem0).wait()
    pltpu.sync_copy(r0, acc.at[dst_v.at[NCHUNK - 2]], add=True)
    pltpu.make_async_copy(hs_hbm.at[s1.at[0]], r1, gsem1).wait()
    pltpu.sync_copy(r1, acc.at[dst_v.at[NCHUNK - 1]], add=True)

    plsc.subcore_barrier()
    pltpu.sync_copy(acc.at[pl.ds(s * RPT, RPT)],
                    out_hbm.at[c, pl.ds(s * RPT, RPT)])


_agg = pl.kernel(
    _agg_body,
    out_type=jax.ShapeDtypeStruct((NC, NP, D), jnp.float32),
    mesh=_MESH,
    scratch_types=[
        pltpu.VMEM((NCHUNK, CH), jnp.int32),
        pltpu.VMEM((1, CH), jnp.int32),
        pltpu.VMEM((1, CH), jnp.int32),
        pltpu.VMEM((CH, D), jnp.float32),
        pltpu.VMEM((CH, D), jnp.float32),
        pltpu.VMEM_SHARED((NP, D), jnp.float32),
        pltpu.SemaphoreType.DMA,
        pltpu.SemaphoreType.DMA,
        pltpu.SemaphoreType.DMA,
        pltpu.SemaphoreType.DMA,
    ],
)


BR = 512
GRID = NP // BR


def _dot(a, b):
    return jax.lax.dot(a, b, precision=jax.lax.Precision.HIGHEST,
                       preferred_element_type=jnp.float32)


def _dinv_col(degp):
    deg = jnp.sum(degp, axis=(0, 1))
    dinv = lax.rsqrt(deg + 1.0)[None, :]
    eye = (lax.broadcasted_iota(jnp.int32, (BR, BR), 0)
           == lax.broadcasted_iota(jnp.int32, (BR, BR), 1))
    return _dot(eye.astype(jnp.float32) * dinv, jnp.ones((BR, 1), jnp.float32))


def _mm_scale_body(x_ref, w_ref, degp_ref, hs_ref, dinv_ref):
    dinv = _dinv_col(degp_ref[...])
    dinv_ref[...] = dinv
    hs_ref[...] = _dot(x_ref[...], w_ref[...]) * dinv


_mm_scale = pl.pallas_call(
    _mm_scale_body,
    grid=(GRID,),
    in_specs=[
        pl.BlockSpec((BR, D), lambda i: (i, 0)),
        pl.BlockSpec((D, D), lambda i: (0, 0)),
        pl.BlockSpec((NW, 1, BR), lambda i: (0, 0, i)),
    ],
    out_specs=[pl.BlockSpec((BR, D), lambda i: (i, 0)),
               pl.BlockSpec((BR, 1), lambda i: (i, 0))],
    out_shape=[jax.ShapeDtypeStruct((NP, D), jnp.float32),
               jax.ShapeDtypeStruct((NP, 1), jnp.float32)],
)


def _mid_body(accp_ref, hs_ref, dinv_ref, b_ref, w_ref, hs2_ref):
    dinv = dinv_ref[...]
    z = (accp_ref[0] + accp_ref[1] + hs_ref[...]) * dinv + b_ref[...]
    z = jnp.maximum(z, 0.0)
    hs2_ref[...] = _dot(z, w_ref[...]) * dinv


_mid = pl.pallas_call(
    _mid_body,
    grid=(GRID,),
    in_specs=[
        pl.BlockSpec((NC, BR, D), lambda i: (0, i, 0)),
        pl.BlockSpec((BR, D), lambda i: (i, 0)),
        pl.BlockSpec((BR, 1), lambda i: (i, 0)),
        pl.BlockSpec((1, D), lambda i: (0, 0)),
        pl.BlockSpec((D, D), lambda i: (0, 0)),
    ],
    out_specs=pl.BlockSpec((BR, D), lambda i: (i, 0)),
    out_shape=jax.ShapeDtypeStruct((NP, D), jnp.float32),
)


def _final_body(accp_ref, hs_ref, dinv_ref, b_ref, out_ref):
    out_ref[...] = ((accp_ref[0] + accp_ref[1] + hs_ref[...]) * dinv_ref[...]
                    + b_ref[...])


_final = pl.pallas_call(
    _final_body,
    grid=(GRID,),
    in_specs=[
        pl.BlockSpec((NC, BR, D), lambda i: (0, i, 0)),
        pl.BlockSpec((BR, D), lambda i: (i, 0)),
        pl.BlockSpec((BR, 1), lambda i: (i, 0)),
        pl.BlockSpec((1, D), lambda i: (0, 0)),
    ],
    out_specs=pl.BlockSpec((BR, D), lambda i: (i, 0)),
    out_shape=jax.ShapeDtypeStruct((N, D), jnp.float32),
)


def kernel(x, edge_index, W1, b1, W2, b2):
    xp = jnp.pad(x, ((0, NP - N), (0, 0)))
    src4 = edge_index[0].astype(jnp.int32).reshape(NW, NCHUNK, 1, CH)
    dst1 = edge_index[1].astype(jnp.int32)
    dst2 = dst1.reshape(EROWS, CH)
    zeroD = jnp.zeros((RPT, D), jnp.float32)
    b1r = b1.reshape(1, D)
    b2r = b2.reshape(1, D)

    degp = _hist(dst1)
    hs1, dinv = _mm_scale(xp, W1, degp)
    acc1 = _agg(hs1, src4, dst2, zeroD)
    hs2 = _mid(acc1, hs1, dinv, b1r, W2)
    acc2 = _agg(hs2, src4, dst2, zeroD)
    return _final(acc2, hs2, dinv, b2r)
